# baseline (speedup 1.0000x reference)
import functools

import jax
import jax.numpy as jnp
from jax import lax
from jax.experimental import pallas as pl
from jax.experimental.pallas import tpu as pltpu
from jax.experimental.pallas import tpu_sc as plsc

N_CLS = 1000
MAX_TOK = 67
D = 512
SUB = D // 128
PROMPT_LEN = 5
PREFIX_LEN = 5
HEAD = PREFIX_LEN + PROMPT_LEN
MAX_LEN = 77
NW = 32
CLS_PER_W = 32
TOK_PAD = 72
START = MAX_TOK // 2 - PROMPT_LEN // 2
LANES = 16


def _body(tok_hbm, table_hbm, prefix_hbm, out_hbm, idx_v, tok_v, head_v,
          g0, g1, o0, o1):
    gsems = (g0, g1)
    osems = (o0, o1)
    wid = lax.axis_index("s") * 2 + lax.axis_index("c")
    base = wid * CLS_PER_W
    pltpu.sync_copy(tok_hbm.at[pl.ds(base * TOK_PAD, CLS_PER_W * TOK_PAD)],
                    idx_v)
    pltpu.sync_copy(prefix_hbm, head_v.at[0, pl.ds(0, PREFIX_LEN)])
    pltpu.sync_copy(prefix_hbm, head_v.at[1, pl.ds(0, PREFIX_LEN)])

    def gather_copy(slot, b):
        return pltpu.make_async_copy(
            table_hbm.at[idx_v.at[pl.ds(slot * TOK_PAD, TOK_PAD)]],
            tok_v.at[b], gsems[b])

    def head_copy(c, b):
        return pltpu.make_async_copy(
            head_v.at[b, pl.ds(0, HEAD)],
            out_hbm.at[c, pl.ds(0, HEAD)], osems[b])

    def tok_copy(c, b):
        return pltpu.make_async_copy(
            tok_v.at[b, pl.ds(0, MAX_TOK)],
            out_hbm.at[c, pl.ds(HEAD, MAX_TOK)], osems[b])

    gather_copy(0, 0).start()

    def pair(t, carry):
        for b in (0, 1):
            slot = 2 * t + b
            c = base + slot
            cond_prev = (c - 1 < N_CLS) if b else ((t >= 1) & (c - 1 < N_CLS))

            @pl.when(cond_prev)
            def _(c=c, b=b):
                head_copy(c - 1, 1 - b).wait()
                tok_copy(c - 1, 1 - b).wait()

            if b == 0:
                gather_copy(slot + 1, 1 - b).start()
            else:
                @pl.when(t < CLS_PER_W // 2 - 1)
                def _(slot=slot, b=b):
                    gather_copy(slot + 1, 1 - b).start()

            gather_copy(slot, b).wait()
            for j in range(D // LANES):
                s, o = j // (128 // LANES), (j % (128 // LANES)) * LANES
                col = pl.ds(o, LANES)
                xs = [tok_v[b, START + k, s, col] for k in range(PROMPT_LEN)]
                mean = (xs[0] + xs[1] + xs[2] + xs[3] + xs[4]) * 0.2
                dfs = [x - mean for x in xs]
                var = (dfs[0] * dfs[0] + dfs[1] * dfs[1] + dfs[2] * dfs[2]
                       + dfs[3] * dfs[3] + dfs[4] * dfs[4]) * 0.25
                yi = jnp.int32(0x5F3759DF) - (
                    lax.bitcast_convert_type(var, jnp.int32) >> 1)
                y = lax.bitcast_convert_type(yi, jnp.float32)
                y = y * (1.5 - 0.5 * var * y * y)
                y = y * (1.5 - 0.5 * var * y * y)
                y = y * (1.5 - 0.5 * var * y * y)
                std = var * y
                scale = 1.0 / (std + 1e-6)
                for k in range(PROMPT_LEN):
                    head_v[b, PREFIX_LEN + k, s, col] = dfs[k] * scale

            @pl.when(c < N_CLS)
            def _(c=c, b=b):
                head_copy(c, b).start()
                tok_copy(c, b).start()

        return carry

    lax.fori_loop(0, CLS_PER_W // 2, pair, 0)
    last = base + CLS_PER_W - 1

    @pl.when(last < N_CLS)
    def _():
        head_copy(last, 1).wait()
        tok_copy(last, 1).wait()


_sc_call = functools.partial(
    pl.kernel,
    out_type=jax.ShapeDtypeStruct((N_CLS, MAX_LEN, SUB, 128), jnp.float32),
    mesh=plsc.VectorSubcoreMesh(core_axis_name="c", subcore_axis_name="s"),
    scratch_types=[
        pltpu.VMEM((CLS_PER_W * TOK_PAD,), jnp.int32),
        pltpu.VMEM((2, TOK_PAD, SUB, 128), jnp.float32),
        pltpu.VMEM((2, 16, SUB, 128), jnp.float32),
        pltpu.SemaphoreType.DMA,
        pltpu.SemaphoreType.DMA,
        pltpu.SemaphoreType.DMA,
        pltpu.SemaphoreType.DMA,
    ],
)(_body)


def kernel(token_ids, table, prefix):
    tok_p = jnp.zeros((NW * CLS_PER_W, TOK_PAD), jnp.int32)
    tok_p = tok_p.at[:N_CLS, :MAX_TOK].set(token_ids.astype(jnp.int32))
    out = _sc_call(tok_p.reshape(-1),
                   table.reshape(table.shape[0], SUB, 128),
                   prefix.reshape(PREFIX_LEN, SUB, 128))
    return out.reshape(N_CLS, MAX_LEN, D)


# speedup vs baseline: 1.1307x; 1.1307x over previous
import functools

import jax
import jax.numpy as jnp
from jax import lax
from jax.experimental import pallas as pl
from jax.experimental.pallas import tpu as pltpu
from jax.experimental.pallas import tpu_sc as plsc

N_CLS = 1000
MAX_TOK = 67
D = 512
SUB = D // 128
PROMPT_LEN = 5
PREFIX_LEN = 5
HEAD = PREFIX_LEN + PROMPT_LEN
MAX_LEN = 77
NW = 32
CLS_PER_W = 32
TOK_PAD = 72
ROWS = HEAD + TOK_PAD
START = MAX_TOK // 2 - PROMPT_LEN // 2
LANES = 16


def _body(tok_hbm, table_hbm, prefix_hbm, out_hbm, idx_v, buf_v, gsem, osem):
    wid = lax.axis_index("s") * 2 + lax.axis_index("c")
    base = wid * CLS_PER_W
    pltpu.sync_copy(tok_hbm.at[pl.ds(base * TOK_PAD, CLS_PER_W * TOK_PAD)],
                    idx_v)
    pltpu.sync_copy(prefix_hbm, buf_v.at[pl.ds(0, PREFIX_LEN)])

    def step(i, carry):
        c = base + i

        @pl.when(c < N_CLS)
        def _():
            pltpu.async_copy(
                table_hbm.at[idx_v.at[pl.ds(i * TOK_PAD, TOK_PAD)]],
                buf_v.at[pl.ds(HEAD, TOK_PAD)], gsem,
            ).wait()
            for j in range(D // LANES):
                s, o = j // (128 // LANES), (j % (128 // LANES)) * LANES
                col = pl.ds(o, LANES)
                xs = [buf_v[HEAD + START + k, s, col]
                      for k in range(PROMPT_LEN)]
                mean = (xs[0] + xs[1] + xs[2] + xs[3] + xs[4]) * 0.2
                dfs = [x - mean for x in xs]
                var = (dfs[0] * dfs[0] + dfs[1] * dfs[1] + dfs[2] * dfs[2]
                       + dfs[3] * dfs[3] + dfs[4] * dfs[4]) * 0.25
                yi = jnp.int32(0x5F3759DF) - (
                    lax.bitcast_convert_type(var, jnp.int32) >> 1)
                y = lax.bitcast_convert_type(yi, jnp.float32)
                y = y * (1.5 - 0.5 * var * y * y)
                y = y * (1.5 - 0.5 * var * y * y)
                y = y * (1.5 - 0.5 * var * y * y)
                std = var * y
                scale = 1.0 / (std + 1e-6)
                for k in range(PROMPT_LEN):
                    buf_v[PREFIX_LEN + k, s, col] = dfs[k] * scale
            pltpu.async_copy(
                buf_v.at[pl.ds(0, MAX_LEN)], out_hbm.at[c], osem,
            ).wait()

        return carry

    lax.fori_loop(0, CLS_PER_W, step, 0)


_sc_call = functools.partial(
    pl.kernel,
    out_type=jax.ShapeDtypeStruct((N_CLS, MAX_LEN, SUB, 128), jnp.float32),
    mesh=plsc.VectorSubcoreMesh(core_axis_name="c", subcore_axis_name="s"),
    scratch_types=[
        pltpu.VMEM((CLS_PER_W * TOK_PAD,), jnp.int32),
        pltpu.VMEM((ROWS, SUB, 128), jnp.float32),
        pltpu.SemaphoreType.DMA,
        pltpu.SemaphoreType.DMA,
    ],
)(_body)


def kernel(token_ids, table, prefix):
    tok_p = jnp.zeros((NW * CLS_PER_W, TOK_PAD), jnp.int32)
    tok_p = tok_p.at[:N_CLS, :MAX_TOK].set(token_ids.astype(jnp.int32))
    out = _sc_call(tok_p.reshape(-1),
                   table.reshape(table.shape[0], SUB, 128),
                   prefix.reshape(PREFIX_LEN, SUB, 128))
    return out.reshape(N_CLS, MAX_LEN, D)


# speedup vs baseline: 2.4861x; 2.1988x over previous
import functools

import jax
import jax.numpy as jnp
from jax import lax
from jax.experimental import pallas as pl
from jax.experimental.pallas import tpu as pltpu
from jax.experimental.pallas import tpu_sc as plsc

N_CLS = 1000
MAX_TOK = 67
D = 512
PROMPT_LEN = 5
PREFIX_LEN = 5
HEAD = PREFIX_LEN + PROMPT_LEN
MAX_LEN = 77
NW = 32
CLS_PER_W = 32
SHIFT = 2
MAIN = SHIFT + MAX_TOK
TOK_PAD = 72
START = MAX_TOK // 2 - PROMPT_LEN // 2
LANES = 16


def _body(tok_hbm, table_hbm, prefix_hbm, out_hbm, idx_v, main_v, head_v,
          gsem, osem):
    wid = lax.axis_index("s") * 2 + lax.axis_index("c")
    base = wid * CLS_PER_W
    pltpu.sync_copy(tok_hbm.at[pl.ds(base * TOK_PAD, CLS_PER_W * TOK_PAD)],
                    idx_v)
    pltpu.sync_copy(prefix_hbm, head_v)

    def step(i, carry):
        c = base + i

        @pl.when(c < N_CLS)
        def _():
            pltpu.async_copy(
                table_hbm.at[idx_v.at[pl.ds(i * TOK_PAD, MAIN)]],
                main_v, gsem,
            ).wait()
            for j in range(D // LANES):
                col = pl.ds(j * LANES, LANES)
                xs = [main_v[SHIFT + START + k, col]
                      for k in range(PROMPT_LEN)]
                mean = (xs[0] + xs[1] + xs[2] + xs[3] + xs[4]) * 0.2
                dfs = [x - mean for x in xs]
                var = (dfs[0] * dfs[0] + dfs[1] * dfs[1] + dfs[2] * dfs[2]
                       + dfs[3] * dfs[3] + dfs[4] * dfs[4]) * 0.25
                yi = jnp.int32(0x5F3759DF) - (
                    lax.bitcast_convert_type(var, jnp.int32) >> 1)
                y = lax.bitcast_convert_type(yi, jnp.float32)
                y = y * (1.5 - 0.5 * var * y * y)
                y = y * (1.5 - 0.5 * var * y * y)
                y = y * (1.5 - 0.5 * var * y * y)
                std = var * y
                scale = 1.0 / (std + 1e-6)
                for k in range(PROMPT_LEN):
                    if k < 3:
                        head_v[PREFIX_LEN + k, col] = dfs[k] * scale
                    else:
                        main_v[k - 3, col] = dfs[k] * scale
            pltpu.async_copy(
                head_v, out_hbm.at[c, pl.ds(0, 8)], osem,
            ).wait()
            pltpu.async_copy(
                main_v, out_hbm.at[c, pl.ds(8, MAIN)], osem,
            ).wait()

        return carry

    lax.fori_loop(0, CLS_PER_W, step, 0)


_sc_call = functools.partial(
    pl.kernel,
    out_type=jax.ShapeDtypeStruct((N_CLS, MAX_LEN, D), jnp.float32),
    mesh=plsc.VectorSubcoreMesh(core_axis_name="c", subcore_axis_name="s"),
    scratch_types=[
        pltpu.VMEM((CLS_PER_W * TOK_PAD,), jnp.int32),
        pltpu.VMEM((MAIN, D), jnp.float32),
        pltpu.VMEM((8, D), jnp.float32),
        pltpu.SemaphoreType.DMA,
        pltpu.SemaphoreType.DMA,
    ],
)(_body)


def kernel(token_ids, table, prefix):
    tok_p = jnp.zeros((NW * CLS_PER_W, TOK_PAD), jnp.int32)
    tok_p = tok_p.at[:N_CLS, SHIFT:SHIFT + MAX_TOK].set(
        token_ids.astype(jnp.int32))
    pref_p = jnp.zeros((8, D), jnp.float32).at[:PREFIX_LEN].set(prefix)
    return _sc_call(tok_p.reshape(-1), table, pref_p)


# speedup vs baseline: 6.2527x; 2.5150x over previous
import functools

import jax
import jax.numpy as jnp
from jax import lax
from jax.experimental import pallas as pl
from jax.experimental.pallas import tpu as pltpu
from jax.experimental.pallas import tpu_sc as plsc

N_CLS = 1000
MAX_TOK = 67
D = 512
PROMPT_LEN = 5
PREFIX_LEN = 5
HEAD = PREFIX_LEN + PROMPT_LEN
MAX_LEN = 77
NW = 32
BLK = 8
NBLK_PAD = 128
BPW = NBLK_PAD // NW
IDX_PER_BLK = MAX_TOK * BLK
START = MAX_TOK // 2 - PROMPT_LEN // 2
LANES = 16
CHUNKS = ((0, 16), (16, 28), (28, 44), (44, 60), (60, 67))
CTX_CHUNK = 2


def _body(idx_hbm, table_hbm, prefix_hbm, out_hbm, idx_v, gbuf, hbuf, pbuf,
          gsem, osem):
    wid = lax.axis_index("s") * 2 + lax.axis_index("c")
    pltpu.sync_copy(idx_hbm.at[pl.ds(wid * BPW * IDX_PER_BLK,
                                     BPW * IDX_PER_BLK)], idx_v)
    pltpu.sync_copy(prefix_hbm, pbuf)

    def fill_prefix(p, carry):
        for j16 in range(D // LANES):
            col = pl.ds(j16 * LANES, LANES)
            v = pbuf[p, col]
            for j in range(BLK):
                hbuf[p * BLK + j, col] = v
        return carry

    lax.fori_loop(0, PREFIX_LEN, fill_prefix, 0, unroll=False)

    def block_step(i, carry):
        blk = wid * BPW + i
        c0 = blk * BLK

        @pl.when(blk < N_CLS // BLK)
        def _():
            ibase = i * IDX_PER_BLK
            for k, (p0, p1) in enumerate(CHUNKS):
                n = (p1 - p0) * BLK
                pltpu.async_copy(
                    table_hbm.at[idx_v.at[pl.ds(ibase + p0 * BLK, n)]],
                    gbuf.at[pl.ds(0, n)], gsem,
                ).wait()

                pend = []
                if k == CTX_CHUNK:
                    def ctx_step(j, carry2):
                        r0 = (START - p0) * BLK + j
                        for j16 in range(D // LANES):
                            col = pl.ds(j16 * LANES, LANES)
                            xs = [gbuf[r0 + s * BLK, col]
                                  for s in range(PROMPT_LEN)]
                            mean = (xs[0] + xs[1] + xs[2] + xs[3]
                                    + xs[4]) * 0.2
                            dfs = [x - mean for x in xs]
                            var = (dfs[0] * dfs[0] + dfs[1] * dfs[1]
                                   + dfs[2] * dfs[2] + dfs[3] * dfs[3]
                                   + dfs[4] * dfs[4]) * 0.25
                            yi = jnp.int32(0x5F3759DF) - (
                                lax.bitcast_convert_type(var, jnp.int32) >> 1)
                            y = lax.bitcast_convert_type(yi, jnp.float32)
                            y = y * (1.5 - 0.5 * var * y * y)
                            y = y * (1.5 - 0.5 * var * y * y)
                            y = y * (1.5 - 0.5 * var * y * y)
                            std = var * y
                            scale = 1.0 / (std + 1e-6)
                            for s in range(PROMPT_LEN):
                                hbuf[(PREFIX_LEN + s) * BLK + j, col] = (
                                    dfs[s] * scale)
                        return carry2

                    lax.fori_loop(0, BLK, ctx_step, 0, unroll=False)
                    for p in range(HEAD):
                        pend.append(pltpu.async_copy(
                            hbuf.at[pl.ds(p * BLK, BLK)],
                            out_hbm.at[p, pl.ds(c0, BLK)], osem))

                for p in range(p0, p1):
                    pend.append(pltpu.async_copy(
                        gbuf.at[pl.ds((p - p0) * BLK, BLK)],
                        out_hbm.at[HEAD + p, pl.ds(c0, BLK)], osem))
                for cp in pend:
                    cp.wait()

        return carry

    lax.fori_loop(0, BPW, block_step, 0, unroll=False)


_sc_call = functools.partial(
    pl.kernel,
    out_type=jax.ShapeDtypeStruct((MAX_LEN, N_CLS, D), jnp.float32),
    mesh=plsc.VectorSubcoreMesh(core_axis_name="c", subcore_axis_name="s"),
    scratch_types=[
        pltpu.VMEM((BPW * IDX_PER_BLK,), jnp.int32),
        pltpu.VMEM((16 * BLK, D), jnp.float32),
        pltpu.VMEM((HEAD * BLK, D), jnp.float32),
        pltpu.VMEM((8, D), jnp.float32),
        pltpu.SemaphoreType.DMA,
        pltpu.SemaphoreType.DMA,
    ],
)(_body)


def kernel(token_ids, table, prefix):
    tok_p = jnp.zeros((NBLK_PAD * BLK, MAX_TOK), jnp.int32)
    tok_p = tok_p.at[:N_CLS].set(token_ids.astype(jnp.int32))
    idx = tok_p.reshape(NBLK_PAD, BLK, MAX_TOK).transpose(0, 2, 1).reshape(-1)
    pref_p = jnp.zeros((8, D), jnp.float32).at[:PREFIX_LEN].set(prefix)
    out = _sc_call(idx, table, pref_p)
    return jnp.transpose(out, (1, 0, 2))


# speedup vs baseline: 6.9629x; 1.1136x over previous
import functools

import jax
import jax.numpy as jnp
from jax import lax
from jax.experimental import pallas as pl
from jax.experimental.pallas import tpu as pltpu
from jax.experimental.pallas import tpu_sc as plsc

N_CLS = 1000
MAX_TOK = 67
D = 512
PROMPT_LEN = 5
PREFIX_LEN = 5
HEAD = PREFIX_LEN + PROMPT_LEN
MAX_LEN = 77
NW = 32
BLK = 8
NBLK = N_CLS // BLK
NBLK_PAD = 128
BPW = NBLK_PAD // NW
IDX_PER_BLK = MAX_TOK * BLK
START = MAX_TOK // 2 - PROMPT_LEN // 2
LANES = 16
CHUNKS = ((0, 10), (10, 20), (20, 30), (30, 40), (40, 50), (50, 60),
          (60, 67))
CTX_CHUNK = 3


def _body(idx_hbm, table_hbm, prefix_hbm, out_hbm, idx_v, gbuf, hbuf, pbuf,
          g0, g1, o0, o1):
    gsems = (g0, g1)
    osems = (o0, o1)
    wid = lax.axis_index("s") * 2 + lax.axis_index("c")
    pltpu.sync_copy(idx_hbm.at[pl.ds(wid * BPW * IDX_PER_BLK,
                                     BPW * IDX_PER_BLK)], idx_v)
    pltpu.sync_copy(prefix_hbm, pbuf)

    def fill_prefix(p, carry):
        for j16 in range(D // LANES):
            col = pl.ds(j16 * LANES, LANES)
            v = pbuf[p, col]
            for j in range(BLK):
                hbuf[p * BLK + j, col] = v
        return carry

    lax.fori_loop(0, PREFIX_LEN, fill_prefix, 0, unroll=False)

    def block_step(i, carry):
        blk = wid * BPW + i
        c0 = blk * BLK

        @pl.when(blk < NBLK)
        def _():
            ibase = i * IDX_PER_BLK

            def gather(k):
                p0, p1 = CHUNKS[k]
                n = (p1 - p0) * BLK
                return pltpu.async_copy(
                    table_hbm.at[idx_v.at[pl.ds(ibase + p0 * BLK, n)]],
                    gbuf.at[k % 2, pl.ds(0, n)], gsems[k % 2])

            g_pend = {0: gather(0)}
            s_pend = {0: [], 1: []}
            for k, (p0, p1) in enumerate(CHUNKS):
                b = k % 2
                g_pend[k].wait()
                if k + 1 < len(CHUNKS):
                    for cp in s_pend[1 - b]:
                        cp.wait()
                    s_pend[1 - b] = []
                    g_pend[k + 1] = gather(k + 1)
                if k == CTX_CHUNK:
                    def ctx_step(j, carry2):
                        r0 = (START - p0) * BLK + j
                        for j16 in range(D // LANES):
                            col = pl.ds(j16 * LANES, LANES)
                            xs = [gbuf[b, r0 + s * BLK, col]
                                  for s in range(PROMPT_LEN)]
                            mean = (xs[0] + xs[1] + xs[2] + xs[3]
                                    + xs[4]) * 0.2
                            dfs = [x - mean for x in xs]
                            var = (dfs[0] * dfs[0] + dfs[1] * dfs[1]
                                   + dfs[2] * dfs[2] + dfs[3] * dfs[3]
                                   + dfs[4] * dfs[4]) * 0.25
                            yi = jnp.int32(0x5F3759DF) - (
                                lax.bitcast_convert_type(var, jnp.int32) >> 1)
                            y = lax.bitcast_convert_type(yi, jnp.float32)
                            y = y * (1.5 - 0.5 * var * y * y)
                            y = y * (1.5 - 0.5 * var * y * y)
                            y = y * (1.5 - 0.5 * var * y * y)
                            std = var * y
                            scale = 1.0 / (std + 1e-6)
                            for s in range(PROMPT_LEN):
                                hbuf[(PREFIX_LEN + s) * BLK + j, col] = (
                                    dfs[s] * scale)
                        return carry2

                    lax.fori_loop(0, BLK, ctx_step, 0, unroll=False)
                    for p in range(HEAD):
                        s_pend[b].append(pltpu.async_copy(
                            hbuf.at[pl.ds(p * BLK, BLK)],
                            out_hbm.at[p, pl.ds(c0, BLK)], osems[b]))
                for p in range(p0, p1):
                    s_pend[b].append(pltpu.async_copy(
                        gbuf.at[b, pl.ds((p - p0) * BLK, BLK)],
                        out_hbm.at[HEAD + p, pl.ds(c0, BLK)], osems[b]))
            for b in (0, 1):
                for cp in s_pend[b]:
                    cp.wait()

        return carry

    lax.fori_loop(0, BPW, block_step, 0, unroll=False)


_sc_call = functools.partial(
    pl.kernel,
    out_type=jax.ShapeDtypeStruct((MAX_LEN, N_CLS, D), jnp.float32),
    mesh=plsc.VectorSubcoreMesh(core_axis_name="c", subcore_axis_name="s"),
    scratch_types=[
        pltpu.VMEM((BPW * IDX_PER_BLK,), jnp.int32),
        pltpu.VMEM((2, 10 * BLK, D), jnp.float32),
        pltpu.VMEM((HEAD * BLK, D), jnp.float32),
        pltpu.VMEM((8, D), jnp.float32),
        pltpu.SemaphoreType.DMA,
        pltpu.SemaphoreType.DMA,
        pltpu.SemaphoreType.DMA,
        pltpu.SemaphoreType.DMA,
    ],
)(_body)


def kernel(token_ids, table, prefix):
    tok_p = jnp.zeros((NBLK_PAD * BLK, MAX_TOK), jnp.int32)
    tok_p = tok_p.at[:N_CLS].set(token_ids.astype(jnp.int32))
    idx = tok_p.reshape(NBLK_PAD, BLK, MAX_TOK).transpose(0, 2, 1).reshape(-1)
    pref_p = jnp.zeros((8, D), jnp.float32).at[:PREFIX_LEN].set(prefix)
    out = _sc_call(idx, table, pref_p)
    return jnp.transpose(out, (1, 0, 2))
